# baseline (speedup 1.0000x reference)
import functools

import jax
import jax.numpy as jnp
from jax import lax
from jax.experimental import pallas as pl
from jax.experimental.pallas import tpu as pltpu
from jax.experimental.pallas import tpu_sc as plsc

NC = 2
NS = 16
L = 16
C = 128


def _spmm_body(n_rows, t0, t1, x_hbm, col_hbm, row_hbm, val_hbm, out_hbm,
               col_v, row_v, val_v, rows_v, acc, sem_i, sem_g, sem_s):
    c = lax.axis_index("c")
    s = lax.axis_index("s")
    f = x_hbm.shape[1]
    nj = f // L
    zrows = rows_v.shape[1]

    stripe = -(-(n_rows // NS) // 16) * 16
    start = s * stripe
    nrows = jnp.minimum(stripe, n_rows - start)

    zero = jnp.zeros((L,), jnp.float32)

    def zb(i, _):
        for j in range(nj):
            rows_v[0, i, pl.ds(j * L, L)] = zero
        return 0

    nbig = nrows // zrows
    tail0 = start + nbig * zrows
    ntail = (nrows - nbig * zrows) // 16

    with jax.named_scope("zero_acc"):
        lax.fori_loop(0, zrows, zb, 0)

        def zcp(i, _):
            pltpu.sync_copy(rows_v.at[0], acc.at[pl.ds(start + i * zrows, zrows)])
            return 0

        lax.fori_loop(0, nbig, zcp, 0)

        def zcp16(i, _):
            pltpu.sync_copy(rows_v.at[0, pl.ds(0, 16)],
                            acc.at[pl.ds(tail0 + i * 16, 16)])
            return 0

        lax.fori_loop(0, ntail, zcp16, 0)
        plsc.subcore_barrier()

    base = jnp.where(c == 0, s * t0, NS * t0 + s * t1) * C
    nt = jnp.where(c == 0, t0, t1)

    def start_idx(t):
        b = lax.rem(t, 3)
        off = base + t * C
        pltpu.make_async_copy(col_hbm.at[pl.ds(off, C)], col_v.at[b], sem_i.at[b]).start()
        pltpu.make_async_copy(row_hbm.at[pl.ds(off, C)], row_v.at[b], sem_i.at[b]).start()
        pltpu.make_async_copy(val_hbm.at[pl.ds(off, C)], val_v.at[b], sem_i.at[b]).start()

    def wait_idx(t):
        b = lax.rem(t, 3)
        pltpu.make_async_copy(col_hbm.at[pl.ds(base, C)], col_v.at[b], sem_i.at[b]).wait()
        pltpu.make_async_copy(row_hbm.at[pl.ds(base, C)], row_v.at[b], sem_i.at[b]).wait()
        pltpu.make_async_copy(val_hbm.at[pl.ds(base, C)], val_v.at[b], sem_i.at[b]).wait()

    def start_gather(t):
        b = lax.rem(t, 3)
        pltpu.make_async_copy(x_hbm.at[col_v.at[b]], rows_v.at[b], sem_g.at[b]).start()

    def wait_gather(t):
        b = lax.rem(t, 3)
        pltpu.make_async_copy(x_hbm.at[col_v.at[b]], rows_v.at[b], sem_g.at[b]).wait()

    def start_scatter(t):
        b = lax.rem(t, 3)
        pltpu.async_copy(rows_v.at[b], acc.at[row_v.at[b]], sem_s.at[b], add=True)

    def wait_scatter(t):
        b = lax.rem(t, 3)
        pltpu.make_async_copy(rows_v.at[b], acc.at[row_v.at[b]], sem_s.at[b]).wait()

    sco = jax.named_scope("edge_loop")
    sco.__enter__()
    start_idx(0)
    start_idx(1)
    wait_idx(0)
    start_gather(0)

    def chunk(t, _):
        b = lax.rem(t, 3)
        wait_gather(t)

        @pl.when(t + 1 < nt)
        def _():
            wait_idx(t + 1)
            start_gather(t + 1)

        @plsc.parallel_loop(0, C // L, 1, unroll=2)
        def scale(g):
            vg = val_v[b, pl.ds(g * L, L)]
            for l in range(L):
                e = g * L + l
                vv = vg[l]
                segs = [rows_v[b, e, pl.ds(j * L, L)] * vv for j in range(nj)]
                for j in range(nj):
                    rows_v[b, e, pl.ds(j * L, L)] = segs[j]

        @pl.when(t >= 1)
        def _():
            wait_scatter(t - 1)

        start_scatter(t)

        @pl.when(t + 2 < nt)
        def _():
            start_idx(t + 2)

        return 0

    lax.fori_loop(0, nt, chunk, 0)
    wait_scatter(nt - 1)
    sco.__exit__(None, None, None)

    def _rne16(u):
        return u + 0x7FFF + (lax.shift_right_logical(u, 16) & 1)

    def pack_rows(src_b, dst_off, m):
        @plsc.parallel_loop(0, m // 2, 1, unroll=2)
        def _(i):
            for j in range(nj):
                a = rows_v[src_b, 2 * i, pl.ds(L * j, L)]
                bq = rows_v[src_b, 2 * i + 1, pl.ds(L * j, L)]
                ua = plsc.bitcast(a, jnp.int32)
                ub = plsc.bitcast(bq, jnp.int32)
                lo = lax.shift_right_logical(_rne16(ua), 16)
                hi = _rne16(ub) & jnp.int32(-65536)
                rows_v[2, dst_off + i, pl.ds(L * j, L)] = plsc.bitcast(
                    lo | hi, jnp.float32)

    with jax.named_scope("writeout"):
        plsc.subcore_barrier()

        start2 = s * (stripe // 2)
        zh = zrows // 2

        def w_copy(i, b):
            return pltpu.make_async_copy(
                rows_v.at[2, pl.ds(b * zh, zh)],
                out_hbm.at[c, pl.ds(start2 + i * zh, zh)], sem_s.at[b])

        def wcp(i, _):
            b = lax.rem(i, 2)
            o = start + i * zrows
            pltpu.sync_copy(acc.at[pl.ds(o, zrows)], rows_v.at[b])

            @pl.when(i >= 2)
            def _():
                w_copy(i - 2, b).wait()

            pack_rows(b, b * zh, zrows)
            w_copy(i, b).start()
            return 0

        lax.fori_loop(0, nbig, wcp, 0)

        @pl.when(nbig >= 2)
        def _():
            w_copy(nbig - 2, lax.rem(nbig - 2, 2)).wait()

        @pl.when(nbig >= 1)
        def _():
            w_copy(nbig - 1, lax.rem(nbig - 1, 2)).wait()

        tail2 = start2 + nbig * zh

        def wcp16(i, _):
            o = tail0 + i * 16
            pltpu.sync_copy(acc.at[pl.ds(o, 16)], rows_v.at[0, pl.ds(0, 16)])
            pack_rows(0, 0, 16)
            pltpu.sync_copy(rows_v.at[2, pl.ds(0, 8)],
                            out_hbm.at[c, pl.ds(tail2 + i * 8, 8)])
            return 0

        lax.fori_loop(0, ntail, wcp16, 0)


def _spmm_sc(xmat, col, row, val, t0, t1):
    n, f = xmat.shape
    mesh = plsc.VectorSubcoreMesh(core_axis_name="c", subcore_axis_name="s")
    kern = functools.partial(
        pl.kernel,
        mesh=mesh,
        compiler_params=pltpu.CompilerParams(needs_layout_passes=False),
        out_type=jax.ShapeDtypeStruct((NC, n // 2, f), jnp.float32),
        scratch_types=[
            pltpu.VMEM((3, C), jnp.int32),
            pltpu.VMEM((3, C), jnp.int32),
            pltpu.VMEM((3, C), jnp.float32),
            pltpu.VMEM((3, C, f), jnp.float32),
            pltpu.VMEM_SHARED((n, f), jnp.float32),
            pltpu.SemaphoreType.DMA((3,)),
            pltpu.SemaphoreType.DMA((3,)),
            pltpu.SemaphoreType.DMA((3,)),
        ],
    )(functools.partial(_spmm_body, n, t0, t1))
    return kern(xmat, col, row, val)


def _unpack_bf16(x):
    return pltpu.bitcast(x, jnp.bfloat16).astype(jnp.float32)


def _mix1_block(x0_ref, pa_ref, pb_ref, w02_ref, w1_ref, b_ref, x1_ref, acc_ref):
    x1 = _unpack_bf16(pa_ref[0]) + _unpack_bf16(pb_ref[0])
    x1_ref[...] = x1
    acc_ref[...] = (jnp.dot(x0_ref[...], w02_ref[...],
                            preferred_element_type=jnp.float32)
                    + jnp.dot(x1, w1_ref[...],
                              preferred_element_type=jnp.float32)
                    + b_ref[...])


def _mix2_block(acc_ref, qa_ref, qb_ref, w2_ref, out_ref):
    q = _unpack_bf16(qa_ref[0]) + _unpack_bf16(qb_ref[0])
    out_ref[...] = acc_ref[...] + jnp.dot(2.0 * q, w2_ref[...],
                                          preferred_element_type=jnp.float32)


def kernel(x, edge_index, edge_values, weight, bias):
    b, v, fin = x.shape
    fin2, kk, fout = weight.shape
    n = b * v
    x0 = x.reshape(n, fin)

    row = edge_index[0].astype(jnp.int32)
    col = edge_index[1].astype(jnp.int32)
    e = row.shape[0]
    per_sub = NC * NS * C
    total = NC * (-(-e // per_sub))
    t0 = (57 * total) // 100
    t1 = total - t0
    e_pad = NS * C * total
    pad = e_pad - e
    row = jnp.pad(row, (0, pad))
    col = jnp.pad(col, (0, pad))
    val = jnp.pad(edge_values, (0, pad))

    p = _spmm_sc(x0, col, row, val, t0, t1)

    w02 = weight[:, 0, :] - weight[:, 2, :]
    w1 = weight[:, 1, :]
    w2 = weight[:, 2, :]

    rb = 2000
    rbh = rb // 2
    nb = n // rb
    grid = (nb,)
    bias2 = bias.reshape(1, fout)
    x1, acc = pl.pallas_call(
        _mix1_block,
        grid=grid,
        in_specs=[
            pl.BlockSpec((rb, fin), lambda i: (i, 0)),
            pl.BlockSpec((1, rbh, fin), lambda i: (0, i, 0)),
            pl.BlockSpec((1, rbh, fin), lambda i: (1, i, 0)),
            pl.BlockSpec((fin, fout), lambda i: (0, 0)),
            pl.BlockSpec((fin, fout), lambda i: (0, 0)),
            pl.BlockSpec((1, fout), lambda i: (0, 0)),
        ],
        out_specs=[
            pl.BlockSpec((rb, fin), lambda i: (i, 0)),
            pl.BlockSpec((rb, fout), lambda i: (i, 0)),
        ],
        out_shape=[
            jax.ShapeDtypeStruct((n, fin), jnp.float32),
            jax.ShapeDtypeStruct((n, fout), jnp.float32),
        ],
    )(x0, p, p, w02, w1, bias2)

    q = _spmm_sc(x1, col, row, val, t0, t1)

    out = pl.pallas_call(
        _mix2_block,
        grid=grid,
        in_specs=[
            pl.BlockSpec((rb, fout), lambda i: (i, 0)),
            pl.BlockSpec((1, rbh, fin), lambda i: (0, i, 0)),
            pl.BlockSpec((1, rbh, fin), lambda i: (1, i, 0)),
            pl.BlockSpec((fin, fout), lambda i: (0, 0)),
        ],
        out_specs=pl.BlockSpec((rb, fout), lambda i: (i, 0)),
        out_shape=jax.ShapeDtypeStruct((n, fout), jnp.float32),
    )(acc, q, q, w2)

    return out.reshape(b, v, fout)


# speedup vs baseline: 1.0214x; 1.0214x over previous
import functools

import jax
import jax.numpy as jnp
from jax import lax
from jax.experimental import pallas as pl
from jax.experimental.pallas import tpu as pltpu
from jax.experimental.pallas import tpu_sc as plsc

NC = 2
NS = 16
L = 16
C = 128


def _spmm_body(n_rows, t0, t1, x_hbm, col_hbm, row_hbm, val_hbm, out_hbm,
               col_v, row_v, val_v, rows_v, acc, sem_i, sem_g, sem_s):
    c = lax.axis_index("c")
    s = lax.axis_index("s")
    f = x_hbm.shape[1]
    nj = f // L
    zrows = C

    stripe = -(-(n_rows // NS) // 16) * 16
    start = s * stripe
    nrows = jnp.minimum(stripe, n_rows - start)

    zero = jnp.zeros((L,), jnp.float32)

    def zb(i, _):
        for j in range(nj):
            rows_v[i, pl.ds(j * L, L)] = zero
        return 0

    nbig = nrows // zrows
    tail0 = start + nbig * zrows
    ntail = (nrows - nbig * zrows) // 16

    with jax.named_scope("zero_acc"):
        lax.fori_loop(0, zrows, zb, 0)

        def zcp(i, _):
            pltpu.sync_copy(rows_v.at[pl.ds(0, zrows)],
                            acc.at[pl.ds(start + i * zrows, zrows)])
            return 0

        lax.fori_loop(0, nbig, zcp, 0)

        def zcp16(i, _):
            pltpu.sync_copy(rows_v.at[pl.ds(0, 16)],
                            acc.at[pl.ds(tail0 + i * 16, 16)])
            return 0

        lax.fori_loop(0, ntail, zcp16, 0)
        plsc.subcore_barrier()

    base = jnp.where(c == 0, s * t0, NS * t0 + s * t1) * C
    nt = jnp.where(c == 0, t0, t1)

    def start_idx(t):
        b = lax.rem(t, 3)
        off = base + t * C
        pltpu.make_async_copy(col_hbm.at[pl.ds(off, C)], col_v.at[b], sem_i.at[b]).start()
        pltpu.make_async_copy(row_hbm.at[pl.ds(off, C)], row_v.at[b], sem_i.at[b]).start()
        pltpu.make_async_copy(val_hbm.at[pl.ds(off, C)], val_v.at[b], sem_i.at[b]).start()

    def wait_idx(t):
        b = lax.rem(t, 3)
        pltpu.make_async_copy(col_hbm.at[pl.ds(base, C)], col_v.at[b], sem_i.at[b]).wait()
        pltpu.make_async_copy(row_hbm.at[pl.ds(base, C)], row_v.at[b], sem_i.at[b]).wait()
        pltpu.make_async_copy(val_hbm.at[pl.ds(base, C)], val_v.at[b], sem_i.at[b]).wait()

    def start_gather(t):
        b = lax.rem(t, 3)
        pltpu.make_async_copy(x_hbm.at[col_v.at[b]],
                              rows_v.at[pl.ds(b * C, C)], sem_g.at[b]).start()

    def wait_gather(t):
        b = lax.rem(t, 3)
        pltpu.make_async_copy(x_hbm.at[col_v.at[b]],
                              rows_v.at[pl.ds(b * C, C)], sem_g.at[b]).wait()

    def start_scatter(t):
        b = lax.rem(t, 3)
        pltpu.async_copy(rows_v.at[pl.ds(b * C, C)], acc.at[row_v.at[b]],
                         sem_s.at[b], add=True)

    def wait_scatter(t):
        b = lax.rem(t, 3)
        pltpu.make_async_copy(rows_v.at[pl.ds(b * C, C)], acc.at[row_v.at[b]],
                              sem_s.at[b]).wait()

    sco = jax.named_scope("edge_loop")
    sco.__enter__()
    start_idx(0)
    start_idx(1)
    wait_idx(0)
    start_gather(0)

    def chunk(t, _):
        b = lax.rem(t, 3)
        wait_gather(t)

        @pl.when(t + 1 < nt)
        def _():
            wait_idx(t + 1)
            start_gather(t + 1)

        @plsc.parallel_loop(0, C // L, 1, unroll=2)
        def scale(g):
            vg = val_v[b, pl.ds(g * L, L)]
            for l in range(L):
                e = b * C + g * L + l
                vv = vg[l]
                segs = [rows_v[e, pl.ds(j * L, L)] * vv for j in range(nj)]
                for j in range(nj):
                    rows_v[e, pl.ds(j * L, L)] = segs[j]

        @pl.when(t >= 1)
        def _():
            wait_scatter(t - 1)

        start_scatter(t)

        @pl.when(t + 2 < nt)
        def _():
            start_idx(t + 2)

        return 0

    lax.fori_loop(0, nt, chunk, 0)
    wait_scatter(nt - 1)
    sco.__exit__(None, None, None)

    def _rne16(u):
        return u + 0x7FFF + (lax.shift_right_logical(u, 16) & 1)

    def pack_pair(src, dst, i, j):
        a = rows_v[src + 2 * i, pl.ds(L * j, L)]
        bq = rows_v[src + 2 * i + 1, pl.ds(L * j, L)]
        ua = plsc.bitcast(a, jnp.int32)
        ub = plsc.bitcast(bq, jnp.int32)
        lo = lax.shift_right_logical(_rne16(ua), 16)
        hi = _rne16(ub) & jnp.int32(-65536)
        rows_v[dst + i, pl.ds(L * j, L)] = plsc.bitcast(lo | hi, jnp.float32)

    with jax.named_scope("writeout"):
        plsc.subcore_barrier()

        start2 = s * (stripe // 2)
        zh = zrows // 2
        stg = 2 * C

        def wstage(i):
            o = start + i * zrows
            pltpu.sync_copy(acc.at[pl.ds(o, zrows)], rows_v.at[pl.ds(stg, zrows)])

        def wcp(i, _):
            wstage(i)

            @plsc.parallel_loop(0, zh, 1, unroll=2)
            def _(ip):
                for j in range(nj):
                    pack_pair(stg, i * zh, ip, j)
            return 0

        lax.fori_loop(0, nbig - 1, wcp, 0)

        wstage(nbig - 1)

        def lastpack(ip, _):
            for j in range(nj):
                pack_pair(stg, (nbig - 1) * zh, ip, j)
            return 0

        lax.fori_loop(0, zh, lastpack, 0)

        def tailcp(i, _):
            pltpu.sync_copy(acc.at[pl.ds(tail0, 16)], rows_v.at[pl.ds(stg, 16)])

            def tpack(ip, _):
                for j in range(nj):
                    pack_pair(stg, nbig * zh, ip, j)
                return 0

            lax.fori_loop(0, 8, tpack, 0)
            return 0

        lax.fori_loop(0, ntail, tailcp, 0)

        @pl.when(s < NS - 1)
        def _():
            pltpu.sync_copy(rows_v.at[pl.ds(0, stripe // 2)],
                            out_hbm.at[c, pl.ds(start2, stripe // 2)])

        @pl.when(s == NS - 1)
        def _():
            last = (n_rows - (NS - 1) * stripe) // 2
            pltpu.sync_copy(rows_v.at[pl.ds(0, last)],
                            out_hbm.at[c, pl.ds(start2, last)])


def _spmm_sc(xmat, col, row, val, t0, t1):
    n, f = xmat.shape
    mesh = plsc.VectorSubcoreMesh(core_axis_name="c", subcore_axis_name="s")
    kern = functools.partial(
        pl.kernel,
        mesh=mesh,
        compiler_params=pltpu.CompilerParams(needs_layout_passes=False),
        out_type=jax.ShapeDtypeStruct((NC, n // 2, f), jnp.float32),
        scratch_types=[
            pltpu.VMEM((3, C), jnp.int32),
            pltpu.VMEM((3, C), jnp.int32),
            pltpu.VMEM((3, C), jnp.float32),
            pltpu.VMEM((3 * C, f), jnp.float32),
            pltpu.VMEM_SHARED((n, f), jnp.float32),
            pltpu.SemaphoreType.DMA((3,)),
            pltpu.SemaphoreType.DMA((3,)),
            pltpu.SemaphoreType.DMA((3,)),
        ],
    )(functools.partial(_spmm_body, n, t0, t1))
    return kern(xmat, col, row, val)


def _unpack_bf16(x):
    return pltpu.bitcast(x, jnp.bfloat16).astype(jnp.float32)


def _mix1_block(x0_ref, pa_ref, pb_ref, w02_ref, w1_ref, b_ref, x1_ref, acc_ref):
    x1 = _unpack_bf16(pa_ref[0]) + _unpack_bf16(pb_ref[0])
    x1_ref[...] = x1
    acc_ref[...] = (jnp.dot(x0_ref[...], w02_ref[...],
                            preferred_element_type=jnp.float32)
                    + jnp.dot(x1, w1_ref[...],
                              preferred_element_type=jnp.float32)
                    + b_ref[...])


def _mix2_block(acc_ref, qa_ref, qb_ref, w2_ref, out_ref):
    q = _unpack_bf16(qa_ref[0]) + _unpack_bf16(qb_ref[0])
    out_ref[...] = acc_ref[...] + jnp.dot(2.0 * q, w2_ref[...],
                                          preferred_element_type=jnp.float32)


def kernel(x, edge_index, edge_values, weight, bias):
    b, v, fin = x.shape
    fin2, kk, fout = weight.shape
    n = b * v
    x0 = x.reshape(n, fin)

    row = edge_index[0].astype(jnp.int32)
    col = edge_index[1].astype(jnp.int32)
    e = row.shape[0]
    per_sub = NC * NS * C
    total = NC * (-(-e // per_sub))
    t0 = (59 * total) // 100
    t1 = total - t0
    e_pad = NS * C * total
    pad = e_pad - e
    row = jnp.pad(row, (0, pad))
    col = jnp.pad(col, (0, pad))
    val = jnp.pad(edge_values, (0, pad))

    p = _spmm_sc(x0, col, row, val, t0, t1)

    w02 = weight[:, 0, :] - weight[:, 2, :]
    w1 = weight[:, 1, :]
    w2 = weight[:, 2, :]

    rb = 2000
    rbh = rb // 2
    nb = n // rb
    grid = (nb,)
    bias2 = bias.reshape(1, fout)
    x1, acc = pl.pallas_call(
        _mix1_block,
        grid=grid,
        in_specs=[
            pl.BlockSpec((rb, fin), lambda i: (i, 0)),
            pl.BlockSpec((1, rbh, fin), lambda i: (0, i, 0)),
            pl.BlockSpec((1, rbh, fin), lambda i: (1, i, 0)),
            pl.BlockSpec((fin, fout), lambda i: (0, 0)),
            pl.BlockSpec((fin, fout), lambda i: (0, 0)),
            pl.BlockSpec((1, fout), lambda i: (0, 0)),
        ],
        out_specs=[
            pl.BlockSpec((rb, fin), lambda i: (i, 0)),
            pl.BlockSpec((rb, fout), lambda i: (i, 0)),
        ],
        out_shape=[
            jax.ShapeDtypeStruct((n, fin), jnp.float32),
            jax.ShapeDtypeStruct((n, fout), jnp.float32),
        ],
    )(x0, p, p, w02, w1, bias2)

    q = _spmm_sc(x1, col, row, val, t0, t1)

    out = pl.pallas_call(
        _mix2_block,
        grid=grid,
        in_specs=[
            pl.BlockSpec((rb, fout), lambda i: (i, 0)),
            pl.BlockSpec((1, rbh, fin), lambda i: (0, i, 0)),
            pl.BlockSpec((1, rbh, fin), lambda i: (1, i, 0)),
            pl.BlockSpec((fin, fout), lambda i: (0, 0)),
        ],
        out_specs=pl.BlockSpec((rb, fout), lambda i: (i, 0)),
        out_shape=jax.ShapeDtypeStruct((n, fout), jnp.float32),
    )(acc, q, q, w2)

    return out.reshape(b, v, fout)


# speedup vs baseline: 1.1812x; 1.1565x over previous
import functools

import jax
import jax.numpy as jnp
from jax import lax
from jax.experimental import pallas as pl
from jax.experimental.pallas import tpu as pltpu
from jax.experimental.pallas import tpu_sc as plsc

NC = 2
NS = 16
L = 16
C = 128


def _spmm_body(n_rows, t0, t1, x_hbm, col_hbm, row_hbm, val_hbm, out_hbm,
               col_v, row_v, val_v, rows_v, acc, sem_i, sem_g, sem_s):
    c = lax.axis_index("c")
    s = lax.axis_index("s")
    f = x_hbm.shape[1]
    nj = f // L
    zrows = C

    stripe = -(-(n_rows // NS) // 16) * 16
    start = s * stripe
    nrows = jnp.minimum(stripe, n_rows - start)

    zero = jnp.zeros((L,), jnp.float32)

    def zb(i, _):
        for j in range(nj):
            rows_v[i, pl.ds(j * L, L)] = zero
        return 0

    nbig = nrows // zrows
    tail0 = start + nbig * zrows
    ntail = (nrows - nbig * zrows) // 16

    with jax.named_scope("zero_acc"):
        lax.fori_loop(0, zrows, zb, 0)

        def zcp(i, _):
            pltpu.sync_copy(rows_v.at[pl.ds(0, zrows)],
                            acc.at[pl.ds(start + i * zrows, zrows)])
            return 0

        lax.fori_loop(0, nbig, zcp, 0)

        def zcp16(i, _):
            pltpu.sync_copy(rows_v.at[pl.ds(0, 16)],
                            acc.at[pl.ds(tail0 + i * 16, 16)])
            return 0

        lax.fori_loop(0, ntail, zcp16, 0)
        plsc.subcore_barrier()

    base = jnp.where(c == 0, s * t0, NS * t0 + s * t1) * C
    nt = jnp.where(c == 0, t0, t1)

    def start_idx(t):
        b = lax.rem(t, 3)
        off = base + t * C
        pltpu.make_async_copy(col_hbm.at[pl.ds(off, C)], col_v.at[b], sem_i.at[b]).start()
        pltpu.make_async_copy(row_hbm.at[pl.ds(off, C)], row_v.at[b], sem_i.at[b]).start()
        pltpu.make_async_copy(val_hbm.at[pl.ds(off, C)], val_v.at[b], sem_i.at[b]).start()

    def wait_idx(t):
        b = lax.rem(t, 3)
        pltpu.make_async_copy(col_hbm.at[pl.ds(base, C)], col_v.at[b], sem_i.at[b]).wait()
        pltpu.make_async_copy(row_hbm.at[pl.ds(base, C)], row_v.at[b], sem_i.at[b]).wait()
        pltpu.make_async_copy(val_hbm.at[pl.ds(base, C)], val_v.at[b], sem_i.at[b]).wait()

    def start_gather(t):
        b = lax.rem(t, 3)
        pltpu.make_async_copy(x_hbm.at[col_v.at[b]],
                              rows_v.at[pl.ds(b * C, C)], sem_g.at[b]).start()

    def wait_gather(t):
        b = lax.rem(t, 3)
        pltpu.make_async_copy(x_hbm.at[col_v.at[b]],
                              rows_v.at[pl.ds(b * C, C)], sem_g.at[b]).wait()

    def start_scatter(t):
        b = lax.rem(t, 3)
        pltpu.async_copy(rows_v.at[pl.ds(b * C, C)], acc.at[row_v.at[b]],
                         sem_s.at[b], add=True)

    def wait_scatter(t):
        b = lax.rem(t, 3)
        pltpu.make_async_copy(rows_v.at[pl.ds(b * C, C)], acc.at[row_v.at[b]],
                              sem_s.at[b]).wait()

    sco = jax.named_scope("edge_loop")
    sco.__enter__()
    start_idx(0)
    start_idx(1)
    wait_idx(0)
    start_gather(0)

    def chunk(t, _):
        b = lax.rem(t, 3)
        wait_gather(t)

        @pl.when(t + 1 < nt)
        def _():
            wait_idx(t + 1)
            start_gather(t + 1)

        @plsc.parallel_loop(0, C // L, 1, unroll=2)
        def scale(g):
            vg = val_v[b, pl.ds(g * L, L)]
            for l in range(L):
                e = b * C + g * L + l
                vv = vg[l]
                segs = [rows_v[e, pl.ds(j * L, L)] * vv for j in range(nj)]
                for j in range(nj):
                    rows_v[e, pl.ds(j * L, L)] = segs[j]

        @pl.when(t >= 1)
        def _():
            wait_scatter(t - 1)

        start_scatter(t)

        @pl.when(t + 2 < nt)
        def _():
            start_idx(t + 2)

        return 0

    lax.fori_loop(0, nt, chunk, 0)
    wait_scatter(nt - 1)
    sco.__exit__(None, None, None)

    def _rne16(u):
        return u + 0x7FFF + (lax.shift_right_logical(u, 16) & 1)

    def pack_pair(src, dst, i, j):
        a = rows_v[src + 2 * i, pl.ds(L * j, L)]
        bq = rows_v[src + 2 * i + 1, pl.ds(L * j, L)]
        ua = plsc.bitcast(a, jnp.int32)
        ub = plsc.bitcast(bq, jnp.int32)
        lo = lax.shift_right_logical(_rne16(ua), 16)
        hi = _rne16(ub) & jnp.int32(-65536)
        rows_v[dst + i, pl.ds(L * j, L)] = plsc.bitcast(lo | hi, jnp.float32)

    with jax.named_scope("writeout"):
        plsc.subcore_barrier()

        start2 = s * (stripe // 2)
        zh = zrows // 2
        stg = 2 * C

        def wstage(i):
            o = start + i * zrows
            pltpu.sync_copy(acc.at[pl.ds(o, zrows)], rows_v.at[pl.ds(stg, zrows)])

        def wcp(i, _):
            wstage(i)

            @plsc.parallel_loop(0, zh, 1, unroll=2)
            def _(ip):
                for j in range(nj):
                    pack_pair(stg, i * zh, ip, j)
            return 0

        lax.fori_loop(0, nbig - 1, wcp, 0)

        wstage(nbig - 1)

        def lastpack(ip, _):
            for j in range(nj):
                pack_pair(stg, (nbig - 1) * zh, ip, j)
            return 0

        lax.fori_loop(0, zh, lastpack, 0)

        def tailcp(i, _):
            pltpu.sync_copy(acc.at[pl.ds(tail0, 16)], rows_v.at[pl.ds(stg, 16)])

            def tpack(ip, _):
                for j in range(nj):
                    pack_pair(stg, nbig * zh, ip, j)
                return 0

            lax.fori_loop(0, 8, tpack, 0)
            return 0

        lax.fori_loop(0, ntail, tailcp, 0)

        @pl.when(s < NS - 1)
        def _():
            pltpu.sync_copy(rows_v.at[pl.ds(0, stripe // 2)],
                            out_hbm.at[c, pl.ds(start2, stripe // 2)])

        @pl.when(s == NS - 1)
        def _():
            last = (n_rows - (NS - 1) * stripe) // 2
            pltpu.sync_copy(rows_v.at[pl.ds(0, last)],
                            out_hbm.at[c, pl.ds(start2, last)])


def _spmm_sc(xmat, col, row, val, t0, t1):
    n, f = xmat.shape
    mesh = plsc.VectorSubcoreMesh(core_axis_name="c", subcore_axis_name="s")
    kern = functools.partial(
        pl.kernel,
        mesh=mesh,
        compiler_params=pltpu.CompilerParams(needs_layout_passes=False),
        out_type=jax.ShapeDtypeStruct((NC, n // 2, f), jnp.float32),
        scratch_types=[
            pltpu.VMEM((3, C), jnp.int32),
            pltpu.VMEM((3, C), jnp.int32),
            pltpu.VMEM((3, C), jnp.float32),
            pltpu.VMEM((3 * C, f), jnp.float32),
            pltpu.VMEM_SHARED((n, f), jnp.float32),
            pltpu.SemaphoreType.DMA((3,)),
            pltpu.SemaphoreType.DMA((3,)),
            pltpu.SemaphoreType.DMA((3,)),
        ],
    )(functools.partial(_spmm_body, n, t0, t1))
    return kern(xmat, col, row, val)


def _unpack_bf16(x):
    return pltpu.bitcast(x, jnp.bfloat16).astype(jnp.float32)


def _mix1_block(x0_ref, pa_ref, pb_ref, w02_ref, w1_ref, b_ref, x1_ref, acc_ref):
    x1 = _unpack_bf16(pa_ref[0]) + _unpack_bf16(pb_ref[0])
    x1_ref[...] = x1
    acc_ref[...] = (jnp.dot(x0_ref[...], w02_ref[...],
                            preferred_element_type=jnp.float32)
                    + jnp.dot(x1, w1_ref[...],
                              preferred_element_type=jnp.float32)
                    + b_ref[...])


def _mix2_block(acc_ref, qa_ref, qb_ref, w2_ref, out_ref):
    q = _unpack_bf16(qa_ref[0]) + _unpack_bf16(qb_ref[0])
    out_ref[...] = acc_ref[...] + jnp.dot(2.0 * q, w2_ref[...],
                                          preferred_element_type=jnp.float32)


def kernel(x, edge_index, edge_values, weight, bias):
    b, v, fin = x.shape
    fin2, kk, fout = weight.shape
    n = b * v
    x0 = x.reshape(n, fin)

    row = edge_index[0].astype(jnp.int32)
    col = edge_index[1].astype(jnp.int32)
    e = row.shape[0]
    per_sub = NC * NS * C
    total = NC * (-(-e // per_sub))
    t0 = (4 * total) // 5
    t1 = total - t0
    e_pad = NS * C * total
    pad = e_pad - e
    row = jnp.pad(row, (0, pad))
    col = jnp.pad(col, (0, pad))
    val = jnp.pad(edge_values, (0, pad))

    p = _spmm_sc(x0, col, row, val, t0, t1)

    w02 = weight[:, 0, :] - weight[:, 2, :]
    w1 = weight[:, 1, :]
    w2 = weight[:, 2, :]

    rb = 2000
    rbh = rb // 2
    nb = n // rb
    grid = (nb,)
    bias2 = bias.reshape(1, fout)
    x1, acc = pl.pallas_call(
        _mix1_block,
        grid=grid,
        in_specs=[
            pl.BlockSpec((rb, fin), lambda i: (i, 0)),
            pl.BlockSpec((1, rbh, fin), lambda i: (0, i, 0)),
            pl.BlockSpec((1, rbh, fin), lambda i: (1, i, 0)),
            pl.BlockSpec((fin, fout), lambda i: (0, 0)),
            pl.BlockSpec((fin, fout), lambda i: (0, 0)),
            pl.BlockSpec((1, fout), lambda i: (0, 0)),
        ],
        out_specs=[
            pl.BlockSpec((rb, fin), lambda i: (i, 0)),
            pl.BlockSpec((rb, fout), lambda i: (i, 0)),
        ],
        out_shape=[
            jax.ShapeDtypeStruct((n, fin), jnp.float32),
            jax.ShapeDtypeStruct((n, fout), jnp.float32),
        ],
    )(x0, p, p, w02, w1, bias2)

    q = _spmm_sc(x1, col, row, val, t0, t1)

    out = pl.pallas_call(
        _mix2_block,
        grid=grid,
        in_specs=[
            pl.BlockSpec((rb, fout), lambda i: (i, 0)),
            pl.BlockSpec((1, rbh, fin), lambda i: (0, i, 0)),
            pl.BlockSpec((1, rbh, fin), lambda i: (1, i, 0)),
            pl.BlockSpec((fin, fout), lambda i: (0, 0)),
        ],
        out_specs=pl.BlockSpec((rb, fout), lambda i: (i, 0)),
        out_shape=jax.ShapeDtypeStruct((n, fout), jnp.float32),
    )(acc, q, q, w2)

    return out.reshape(b, v, fout)
